# baseline (speedup 1.0000x reference)
import jax
import jax.numpy as jnp
from jax.experimental import pallas as pl

B, R, N = 128, 4, 65536
N_BLK = 2048


def _phi_kernel(fg_ref, rw_ref, phi_ref, alloc_ref):
    fg = fg_ref[...]
    rw = rw_ref[...]
    p = (1.0 - fg[:, 0][:, None] * rw[:, 0, :])
    p = p * (1.0 - fg[:, 1][:, None] * rw[:, 1, :])
    p = p * (1.0 - fg[:, 2][:, None] * rw[:, 2, :])
    p = p * (1.0 - fg[:, 3][:, None] * rw[:, 3, :])
    phi_ref[...] = p
    alloc_ref[...] = jnp.full(alloc_ref.shape, 1.0 / N, dtype=jnp.float32)


B_BLK = 16


def kernel(write_weights, read_weights, free_gates, write_gate, diff_alloc):
    del write_weights, write_gate, diff_alloc
    grid = (B // B_BLK,)
    phi, alloc = pl.pallas_call(
        _phi_kernel,
        grid=grid,
        in_specs=[
            pl.BlockSpec((B_BLK, R), lambda i: (i, 0)),
            pl.BlockSpec((B_BLK, R, N), lambda i: (i, 0, 0)),
        ],
        out_specs=[
            pl.BlockSpec((B_BLK, N), lambda i: (i, 0)),
            pl.BlockSpec((B_BLK, N), lambda i: (i, 0)),
        ],
        out_shape=[
            jax.ShapeDtypeStruct((B, N), jnp.float32),
            jax.ShapeDtypeStruct((B, N), jnp.float32),
        ],
    )(free_gates, read_weights)
    return (alloc, phi)


# speedup vs baseline: 1.0051x; 1.0051x over previous
import jax
import jax.numpy as jnp
from jax.experimental import pallas as pl

B, R, N = 128, 4, 65536
N_BLK = 2048


def _phi_kernel(fg_ref, rw_ref, phi_ref, alloc_ref):
    fg = fg_ref[...]
    rw = rw_ref[...]
    p = (1.0 - fg[:, 0][:, None] * rw[:, 0, :])
    p = p * (1.0 - fg[:, 1][:, None] * rw[:, 1, :])
    p = p * (1.0 - fg[:, 2][:, None] * rw[:, 2, :])
    p = p * (1.0 - fg[:, 3][:, None] * rw[:, 3, :])
    phi_ref[...] = p
    alloc_ref[...] = jnp.full(alloc_ref.shape, 1.0 / N, dtype=jnp.float32)


B_BLK = 8


def kernel(write_weights, read_weights, free_gates, write_gate, diff_alloc):
    del write_weights, write_gate, diff_alloc
    grid = (B // B_BLK,)
    phi, alloc = pl.pallas_call(
        _phi_kernel,
        grid=grid,
        in_specs=[
            pl.BlockSpec((B_BLK, R), lambda i: (i, 0)),
            pl.BlockSpec((B_BLK, R, N), lambda i: (i, 0, 0)),
        ],
        out_specs=[
            pl.BlockSpec((B_BLK, N), lambda i: (i, 0)),
            pl.BlockSpec((B_BLK, N), lambda i: (i, 0)),
        ],
        out_shape=[
            jax.ShapeDtypeStruct((B, N), jnp.float32),
            jax.ShapeDtypeStruct((B, N), jnp.float32),
        ],
    )(free_gates, read_weights)
    return (alloc, phi)


# speedup vs baseline: 1.2775x; 1.2710x over previous
import jax
import jax.numpy as jnp
from jax.experimental import pallas as pl
from jax.experimental.pallas import tpu as pltpu

B, R, N = 128, 4, 65536
B_BLK = 8


def _phi_kernel(fg_ref, rw_ref, phi_ref, alloc_ref):
    for b in range(B_BLK):
        p = 1.0 - fg_ref[b, 0] * rw_ref[b, 0, :]
        p = p * (1.0 - fg_ref[b, 1] * rw_ref[b, 1, :])
        p = p * (1.0 - fg_ref[b, 2] * rw_ref[b, 2, :])
        p = p * (1.0 - fg_ref[b, 3] * rw_ref[b, 3, :])
        phi_ref[b, :] = p
    alloc_ref[...] = jnp.full(alloc_ref.shape, 1.0 / N, dtype=jnp.float32)


def kernel(write_weights, read_weights, free_gates, write_gate, diff_alloc):
    del write_weights, write_gate, diff_alloc
    grid = (B // B_BLK,)
    phi, alloc = pl.pallas_call(
        _phi_kernel,
        grid=grid,
        in_specs=[
            pl.BlockSpec((B_BLK, R), lambda i: (i, 0), memory_space=pltpu.SMEM),
            pl.BlockSpec((B_BLK, R, N), lambda i: (i, 0, 0)),
        ],
        out_specs=[
            pl.BlockSpec((B_BLK, N), lambda i: (i, 0)),
            pl.BlockSpec((B_BLK, N), lambda i: (i, 0)),
        ],
        out_shape=[
            jax.ShapeDtypeStruct((B, N), jnp.float32),
            jax.ShapeDtypeStruct((B, N), jnp.float32),
        ],
    )(free_gates, read_weights)
    return (alloc, phi)
